# Initial kernel scaffold; baseline (speedup 1.0000x reference)
import functools

import jax
import jax.numpy as jnp
from jax import lax
from jax.experimental import pallas as pl
from jax.experimental.pallas import tpu as pltpu
from jax.experimental.pallas import tpu_sc as plsc

N_NODES = 10000
D = 128
N_EDGES = 320000

NC = 2
NS = 16
NW = NC * NS

CH = 128
NCHUNK = N_EDGES // CH
BASE_CH = NCHUNK // NW
EXTRA = NCHUNK - BASE_CH * NW
ROWS_PER_TILE = N_NODES // NS
ZR = 125

_f32 = jnp.float32
_mesh = plsc.VectorSubcoreMesh(core_axis_name="c", subcore_axis_name="s")


def _make_sc_agg(compute_deg: bool):
    out_type = [jax.ShapeDtypeStruct((NC, N_NODES, D), _f32)]
    if compute_deg:
        out_type.append(jax.ShapeDtypeStruct((NC, N_NODES), _f32))

    scratch = [
        pltpu.VMEM((CH,), jnp.int32),
        pltpu.VMEM((CH,), jnp.int32),
        pltpu.VMEM((CH, D), _f32),
        pltpu.VMEM((ZR, D), _f32),
        pltpu.VMEM_SHARED((N_NODES, D), _f32),
        pltpu.SemaphoreType.DMA,
    ]
    if compute_deg:
        scratch += [
            pltpu.VMEM((CH,), _f32),
            pltpu.VMEM((2000,), _f32),
            pltpu.VMEM_SHARED((N_NODES,), _f32),
        ]

    def body(x_hbm, src_hbm, dst_hbm, *rest):
        if compute_deg:
            (agg_out, deg_out, src_v, dst_v, rows_v, zrow_v, agg_sh, sem,
             ones_v, zdeg_v, deg_sh) = rest
        else:
            (agg_out, src_v, dst_v, rows_v, zrow_v, agg_sh, sem) = rest

        cid = lax.axis_index("c")
        sid = lax.axis_index("s")
        wid = cid * NS + sid

        def zrow_body(i, carry):
            for j in range(D // 16):
                zrow_v[i, pl.ds(j * 16, 16)] = jnp.zeros((16,), _f32)
            return carry

        lax.fori_loop(0, ZR, zrow_body, 0)
        for t in range(ROWS_PER_TILE // ZR):
            pltpu.sync_copy(
                zrow_v, agg_sh.at[pl.ds(sid * ROWS_PER_TILE + t * ZR, ZR)])

        if compute_deg:
            def zdeg_body(i, carry):
                zdeg_v[pl.ds(i * 16, 16)] = jnp.zeros((16,), _f32)
                return carry

            lax.fori_loop(0, 2000 // 16, zdeg_body, 0)

            def ones_body(i, carry):
                ones_v[pl.ds(i * 16, 16)] = jnp.ones((16,), _f32)
                return carry

            lax.fori_loop(0, CH // 16, ones_body, 0)

            @pl.when(sid < N_NODES // 2000)
            def _():
                pltpu.sync_copy(zdeg_v, deg_sh.at[pl.ds(sid * 2000, 2000)])

        plsc.subcore_barrier()

        nch = BASE_CH + jnp.where(wid < EXTRA, 1, 0)
        ch0 = BASE_CH * wid + jnp.minimum(wid, EXTRA)

        def chunk_body(i, carry):
            base = pl.multiple_of((ch0 + i) * CH, CH)
            pltpu.sync_copy(src_hbm.at[pl.ds(base, CH)], src_v)
            pltpu.sync_copy(dst_hbm.at[pl.ds(base, CH)], dst_v)
            pltpu.async_copy(x_hbm.at[src_v], rows_v, sem).wait()
            pltpu.sync_copy(rows_v, agg_sh.at[dst_v], add=True)
            if compute_deg:
                pltpu.sync_copy(ones_v, deg_sh.at[dst_v], add=True)
            return carry

        lax.fori_loop(0, nch, chunk_body, 0)

        plsc.subcore_barrier()

        pltpu.sync_copy(
            agg_sh.at[pl.ds(sid * ROWS_PER_TILE, ROWS_PER_TILE)],
            agg_out.at[cid, pl.ds(sid * ROWS_PER_TILE, ROWS_PER_TILE)])
        if compute_deg:
            @pl.when(sid == 0)
            def _():
                pltpu.sync_copy(deg_sh, deg_out.at[cid])

    return pl.kernel(
        body, mesh=_mesh, out_type=tuple(out_type), scratch_types=scratch)


_sc_agg_deg = _make_sc_agg(True)
_sc_agg = _make_sc_agg(False)


BLK = 1000


def _dot_t(a, w):
    return lax.dot_general(a, w, (((1,), (1,)), ((), ())),
                           preferred_element_type=_f32)


def _tc_layer1(aggp_ref, degp_ref, x_ref, wl_ref, wr_ref, b_ref,
               h_ref, inv_ref):
    deg = degp_ref[:, 0] + degp_ref[:, 1]
    inv = 1.0 / jnp.maximum(deg, 1.0)
    agg = (aggp_ref[0] + aggp_ref[1]) * inv[:, None]
    h = (_dot_t(agg, wl_ref[...]) + b_ref[...]
         + _dot_t(x_ref[...], wr_ref[...]))
    h_ref[...] = jnp.maximum(h, 0.0)
    inv_ref[...] = inv[:, None]


def _tc_layer2(aggp_ref, inv_ref, h_ref, wl_ref, wr_ref, b_ref, out_ref):
    agg = (aggp_ref[0] + aggp_ref[1]) * inv_ref[...]
    out_ref[...] = (_dot_t(agg, wl_ref[...]) + b_ref[...]
                    + _dot_t(h_ref[...], wr_ref[...]))


_w_spec = pl.BlockSpec((D, D), lambda i: (0, 0))
_b_spec = pl.BlockSpec((1, D), lambda i: (0, 0))
_aggp_spec = pl.BlockSpec((NC, BLK, D), lambda i: (0, i, 0))
_row_spec = pl.BlockSpec((BLK, D), lambda i: (i, 0))
_inv_spec = pl.BlockSpec((BLK, 1), lambda i: (i, 0))


def _tc1(aggp, degp_t, x, wl, wr, b):
    return pl.pallas_call(
        _tc_layer1,
        grid=(N_NODES // BLK,),
        in_specs=[_aggp_spec,
                  pl.BlockSpec((BLK, NC), lambda i: (i, 0)),
                  _row_spec, _w_spec, _w_spec, _b_spec],
        out_specs=[_row_spec, _inv_spec],
        out_shape=[jax.ShapeDtypeStruct((N_NODES, D), _f32),
                   jax.ShapeDtypeStruct((N_NODES, 1), _f32)],
    )(aggp, degp_t, x, wl, wr, b)


def _tc2(aggp, inv, h, wl, wr, b):
    return pl.pallas_call(
        _tc_layer2,
        grid=(N_NODES // BLK,),
        in_specs=[_aggp_spec, _inv_spec, _row_spec, _w_spec, _w_spec, _b_spec],
        out_specs=_row_spec,
        out_shape=jax.ShapeDtypeStruct((N_NODES, D), _f32),
    )(aggp, inv, h, wl, wr, b)


@jax.jit
def kernel(x, edge_index, W1_l, b1_l, W1_r, W2_l, b2_l, W2_r):
    x = x.astype(_f32)
    src = edge_index[0].astype(jnp.int32)
    dst = edge_index[1].astype(jnp.int32)

    aggp1, degp = _sc_agg_deg(x, src, dst)
    h, inv = _tc1(aggp1, degp.T, x, W1_l, W1_r, b1_l.reshape(1, D))

    (aggp2,) = _sc_agg(h, src, dst)
    out = _tc2(aggp2, inv, h, W2_l, W2_r, b2_l.reshape(1, D))
    return out


# speedup vs baseline: 6.8138x; 6.8138x over previous
import jax
import jax.numpy as jnp
from jax import lax
from jax.experimental import pallas as pl
from jax.experimental.pallas import tpu as pltpu
from jax.experimental.pallas import tpu_sc as plsc

N_NODES = 10000
D = 128
N_EDGES = 320000

NC = 2
NS = 16
NW = NC * NS

CH = 128
NCHUNK = N_EDGES // CH
BASE_CH = NCHUNK // NW
EXTRA = NCHUNK - BASE_CH * NW

RPT = 632
NPAD = NS * RPT

_f32 = jnp.float32
_mesh = plsc.VectorSubcoreMesh(core_axis_name="c", subcore_axis_name="s")


def _make_sc_agg(compute_deg: bool):
    out_type = [jax.ShapeDtypeStruct((NC, NPAD, D), _f32)]
    if compute_deg:
        out_type.append(jax.ShapeDtypeStruct((NC * N_NODES,), _f32))

    scratch = [
        pltpu.VMEM((CH,), jnp.int32),
        pltpu.VMEM((CH,), jnp.int32),
        pltpu.VMEM((CH, D), _f32),
        pltpu.VMEM_SHARED((NPAD, D), _f32),
        pltpu.SemaphoreType.DMA,
    ]
    if compute_deg:
        scratch += [
            pltpu.VMEM((CH,), _f32),
            pltpu.VMEM((N_NODES,), _f32),
            pltpu.VMEM_SHARED((N_NODES,), _f32),
        ]

    def body(x_hbm, src_hbm, dst_hbm, z2d_hbm, z1d_hbm, *rest):
        if compute_deg:
            (agg_out, deg_out, src_v, dst_v, rows_v, agg_sh, sem,
             ones_v, deg_v, deg_sh) = rest
        else:
            (agg_out, src_v, dst_v, rows_v, agg_sh, sem) = rest

        cid = lax.axis_index("c")
        sid = lax.axis_index("s")
        wid = cid * NS + sid

        pltpu.sync_copy(z2d_hbm.at[pl.ds(sid * RPT, RPT)],
                        agg_sh.at[pl.ds(sid * RPT, RPT)])

        if compute_deg:
            @pl.when(sid == 0)
            def _():
                pltpu.sync_copy(z1d_hbm, deg_v)
                pltpu.sync_copy(deg_v, deg_sh)

            def ones_body(i, carry):
                ones_v[pl.ds(i * 16, 16)] = jnp.ones((16,), _f32)
                return carry

            lax.fori_loop(0, CH // 16, ones_body, 0)

        plsc.subcore_barrier()

        nch = BASE_CH + jnp.where(wid < EXTRA, 1, 0)
        ch0 = BASE_CH * wid + jnp.minimum(wid, EXTRA)

        def chunk_body(i, carry):
            base = pl.multiple_of((ch0 + i) * CH, CH)
            pltpu.sync_copy(src_hbm.at[pl.ds(base, CH)], src_v)
            pltpu.sync_copy(dst_hbm.at[pl.ds(base, CH)], dst_v)
            pltpu.async_copy(x_hbm.at[src_v], rows_v, sem).wait()
            pltpu.sync_copy(rows_v, agg_sh.at[dst_v], add=True)
            if compute_deg:
                pltpu.sync_copy(ones_v, deg_sh.at[dst_v], add=True)
            return carry

        lax.fori_loop(0, nch, chunk_body, 0)

        plsc.subcore_barrier()

        pltpu.sync_copy(agg_sh.at[pl.ds(sid * RPT, RPT)],
                        agg_out.at[cid, pl.ds(sid * RPT, RPT)])
        if compute_deg:
            @pl.when(sid == 0)
            def _():
                pltpu.sync_copy(deg_sh, deg_v)
                pltpu.sync_copy(deg_v,
                                deg_out.at[pl.ds(cid * N_NODES, N_NODES)])

    return pl.kernel(
        body, mesh=_mesh, out_type=tuple(out_type), scratch_types=scratch)


_sc_agg_deg = _make_sc_agg(True)
_sc_agg = _make_sc_agg(False)


BLK = 1000


def _dot_t(a, w):
    return lax.dot_general(a, w, (((1,), (1,)), ((), ())),
                           preferred_element_type=_f32)


def _tc_layer1(aggp_ref, degp_ref, x_ref, wl_ref, wr_ref, b_ref,
               h_ref, inv_ref):
    deg = degp_ref[:, 0] + degp_ref[:, 1]
    inv = 1.0 / jnp.maximum(deg, 1.0)
    agg = (aggp_ref[0] + aggp_ref[1]) * inv[:, None]
    h = (_dot_t(agg, wl_ref[...]) + b_ref[...]
         + _dot_t(x_ref[...], wr_ref[...]))
    h_ref[...] = jnp.maximum(h, 0.0)
    inv_ref[...] = inv[:, None]


def _tc_layer2(aggp_ref, inv_ref, h_ref, wl_ref, wr_ref, b_ref, out_ref):
    agg = (aggp_ref[0] + aggp_ref[1]) * inv_ref[...]
    out_ref[...] = (_dot_t(agg, wl_ref[...]) + b_ref[...]
                    + _dot_t(h_ref[...], wr_ref[...]))


_w_spec = pl.BlockSpec((D, D), lambda i: (0, 0))
_b_spec = pl.BlockSpec((1, D), lambda i: (0, 0))
_aggp_spec = pl.BlockSpec((NC, BLK, D), lambda i: (0, i, 0))
_row_spec = pl.BlockSpec((BLK, D), lambda i: (i, 0))
_inv_spec = pl.BlockSpec((BLK, 1), lambda i: (i, 0))


def _tc1(aggp, degp_t, x, wl, wr, b):
    return pl.pallas_call(
        _tc_layer1,
        grid=(N_NODES // BLK,),
        in_specs=[_aggp_spec,
                  pl.BlockSpec((BLK, NC), lambda i: (i, 0)),
                  _row_spec, _w_spec, _w_spec, _b_spec],
        out_specs=[_row_spec, _inv_spec],
        out_shape=[jax.ShapeDtypeStruct((N_NODES, D), _f32),
                   jax.ShapeDtypeStruct((N_NODES, 1), _f32)],
    )(aggp, degp_t, x, wl, wr, b)


def _tc2(aggp, inv, h, wl, wr, b):
    return pl.pallas_call(
        _tc_layer2,
        grid=(N_NODES // BLK,),
        in_specs=[_aggp_spec, _inv_spec, _row_spec, _w_spec, _w_spec, _b_spec],
        out_specs=_row_spec,
        out_shape=jax.ShapeDtypeStruct((N_NODES, D), _f32),
    )(aggp, inv, h, wl, wr, b)


@jax.jit
def kernel(x, edge_index, W1_l, b1_l, W1_r, W2_l, b2_l, W2_r):
    x = x.astype(_f32)
    src = edge_index[0].astype(jnp.int32)
    dst = edge_index[1].astype(jnp.int32)

    z2d = jnp.zeros((NPAD, D), _f32)
    z1d = jnp.zeros((N_NODES,), _f32)

    aggp1, deg_flat = _sc_agg_deg(x, src, dst, z2d, z1d)
    degp_t = deg_flat.reshape(NC, N_NODES).T
    h, inv = _tc1(aggp1, degp_t, x, W1_l, W1_r, b1_l.reshape(1, D))

    (aggp2,) = _sc_agg(h, src, dst, z2d, z1d)
    out = _tc2(aggp2, inv, h, W2_l, W2_r, b2_l.reshape(1, D))
    return out


# speedup vs baseline: 13.2545x; 1.9452x over previous
import jax
import jax.numpy as jnp
from jax import lax
from jax.experimental import pallas as pl
from jax.experimental.pallas import tpu as pltpu
from jax.experimental.pallas import tpu_sc as plsc

N_NODES = 10000
D = 128
N_EDGES = 320000

NC = 2
NS = 16
NW = NC * NS

CH = 128
NITER = 80
NCH_TOT = NW * NITER
E_PAD = NCH_TOT * CH
NB = 2
NPH = 2
PH = NITER // NPH

RPT = 632
NPAD = NS * RPT
DEG_E = PH * CH

_f32 = jnp.float32
_mesh = plsc.VectorSubcoreMesh(core_axis_name="c", subcore_axis_name="s")


def _make_sc_agg(compute_deg: bool):
    out_type = [jax.ShapeDtypeStruct((NC, NPAD, D), _f32)]
    if compute_deg:
        out_type.append(jax.ShapeDtypeStruct((NC * NPAD,), _f32))

    scratch = [
        pltpu.VMEM((PH, CH), jnp.int32),
        pltpu.VMEM((PH, CH), jnp.int32),
        pltpu.VMEM((NB, CH, D), _f32),
        pltpu.VMEM_SHARED((NPAD, D), _f32),
    ] + [pltpu.SemaphoreType.DMA] * (2 * NB)
    if compute_deg:
        scratch += [
            pltpu.VMEM((CH,), _f32),
            pltpu.VMEM((DEG_E,), _f32),
            pltpu.VMEM_SHARED((NPAD,), _f32),
            pltpu.SemaphoreType.DMA,
        ]

    def body(x_hbm, src_hbm, dst_hbm, z2d_hbm, z1d_hbm, *rest):
        if compute_deg:
            (agg_out, deg_out, src_v, dst_v, rows_v, agg_sh,
             *sems) = rest[:6 + 2 * NB]
            ones_v, deg_v, deg_sh, dsem = rest[6 + 2 * NB:]
        else:
            (agg_out, src_v, dst_v, rows_v, agg_sh, *sems) = rest
        gsem = sems[:NB]
        ssem = sems[NB:2 * NB]

        cid = lax.axis_index("c")
        sid = lax.axis_index("s")
        wid = cid * NS + sid

        pltpu.sync_copy(z2d_hbm.at[pl.ds(sid * RPT, RPT)],
                        agg_sh.at[pl.ds(sid * RPT, RPT)])

        if compute_deg:
            @pl.when(sid == 0)
            def _():
                pltpu.sync_copy(z1d_hbm, deg_v)
                pltpu.sync_copy(deg_v, deg_sh.at[pl.ds(0, DEG_E)])
                pltpu.sync_copy(deg_v.at[pl.ds(0, NPAD - DEG_E)],
                                deg_sh.at[pl.ds(DEG_E, NPAD - DEG_E)])

            def ones_body(i, carry):
                ones_v[pl.ds(i * 16, 16)] = jnp.ones((16,), _f32)
                return carry

            lax.fori_loop(0, CH // 16, ones_body, 0)

        plsc.subcore_barrier()

        def wait_bytes(sem, like):
            pltpu.make_async_copy(x_hbm.at[pl.ds(0, CH)], like, sem).wait()

        def wait_deg(sem):
            pltpu.make_async_copy(z1d_hbm, deg_v, sem).wait()

        def step(i, b, refill):
            wait_bytes(gsem[b], rows_v.at[b])
            pltpu.async_copy(rows_v.at[b], agg_sh.at[dst_v.at[i]],
                             ssem[b], add=True)
            if compute_deg:
                pltpu.async_copy(ones_v, deg_sh.at[dst_v.at[i]],
                                 dsem, add=True)
            if refill:
                wait_bytes(ssem[b], rows_v.at[b])
                pltpu.async_copy(x_hbm.at[src_v.at[i + NB]],
                                 rows_v.at[b], gsem[b])

        for ph in range(NPH):
            if ph > 0:
                for b in range(NB):
                    wait_bytes(ssem[b], rows_v.at[b])
                if compute_deg:
                    wait_deg(dsem)

            base = wid * NITER + ph * PH
            pltpu.sync_copy(src_hbm.at[pl.ds(base, PH)], src_v)
            pltpu.sync_copy(dst_hbm.at[pl.ds(base, PH)], dst_v)

            for b in range(NB):
                pltpu.async_copy(x_hbm.at[src_v.at[b]], rows_v.at[b],
                                 gsem[b])

            def group_body(p, carry):
                i0 = p * NB
                for b in range(NB):
                    step(i0 + b, b, refill=True)
                return carry

            lax.fori_loop(0, PH // NB - 1, group_body, 0)
            for b in range(NB):
                step(PH - NB + b, b, refill=False)

        for b in range(NB):
            wait_bytes(ssem[b], rows_v.at[b])
        if compute_deg:
            wait_deg(dsem)

        plsc.subcore_barrier()

        pltpu.sync_copy(agg_sh.at[pl.ds(sid * RPT, RPT)],
                        agg_out.at[cid, pl.ds(sid * RPT, RPT)])
        if compute_deg:
            @pl.when(sid == 0)
            def _():
                rem = NPAD - DEG_E
                pltpu.sync_copy(deg_sh.at[pl.ds(0, DEG_E)], deg_v)
                pltpu.sync_copy(deg_v,
                                deg_out.at[pl.ds(cid * NPAD, DEG_E)])
                pltpu.sync_copy(deg_sh.at[pl.ds(DEG_E, rem)],
                                deg_v.at[pl.ds(0, rem)])
                pltpu.sync_copy(deg_v.at[pl.ds(0, rem)],
                                deg_out.at[pl.ds(cid * NPAD + DEG_E, rem)])

    return pl.kernel(
        body, mesh=_mesh, out_type=tuple(out_type), scratch_types=scratch)


_sc_agg_deg = _make_sc_agg(True)
_sc_agg = _make_sc_agg(False)


BLK = 1000


def _dot_t(a, w):
    return lax.dot_general(a, w, (((1,), (1,)), ((), ())),
                           preferred_element_type=_f32)


def _tc_layer1(aggp_ref, degp_ref, x_ref, wl_ref, wr_ref, b_ref,
               h_ref, inv_ref):
    deg = degp_ref[:, 0] + degp_ref[:, 1]
    inv = 1.0 / jnp.maximum(deg, 1.0)
    agg = (aggp_ref[0] + aggp_ref[1]) * inv[:, None]
    h = (_dot_t(agg, wl_ref[...]) + b_ref[...]
         + _dot_t(x_ref[...], wr_ref[...]))
    h_ref[...] = jnp.maximum(h, 0.0)
    inv_ref[...] = inv[:, None]


def _tc_layer2(aggp_ref, inv_ref, h_ref, wl_ref, wr_ref, b_ref, out_ref):
    agg = (aggp_ref[0] + aggp_ref[1]) * inv_ref[...]
    out_ref[...] = (_dot_t(agg, wl_ref[...]) + b_ref[...]
                    + _dot_t(h_ref[...], wr_ref[...]))


_w_spec = pl.BlockSpec((D, D), lambda i: (0, 0))
_b_spec = pl.BlockSpec((1, D), lambda i: (0, 0))
_aggp_spec = pl.BlockSpec((NC, BLK, D), lambda i: (0, i, 0))
_row_spec = pl.BlockSpec((BLK, D), lambda i: (i, 0))
_inv_spec = pl.BlockSpec((BLK, 1), lambda i: (i, 0))


def _tc1(aggp, degp_t, x, wl, wr, b):
    return pl.pallas_call(
        _tc_layer1,
        grid=(N_NODES // BLK,),
        in_specs=[_aggp_spec,
                  pl.BlockSpec((BLK, NC), lambda i: (i, 0)),
                  _row_spec, _w_spec, _w_spec, _b_spec],
        out_specs=[_row_spec, _inv_spec],
        out_shape=[jax.ShapeDtypeStruct((N_NODES, D), _f32),
                   jax.ShapeDtypeStruct((N_NODES, 1), _f32)],
    )(aggp, degp_t, x, wl, wr, b)


def _tc2(aggp, inv, h, wl, wr, b):
    return pl.pallas_call(
        _tc_layer2,
        grid=(N_NODES // BLK,),
        in_specs=[_aggp_spec, _inv_spec, _row_spec, _w_spec, _w_spec, _b_spec],
        out_specs=_row_spec,
        out_shape=jax.ShapeDtypeStruct((N_NODES, D), _f32),
    )(aggp, inv, h, wl, wr, b)


@jax.jit
def kernel(x, edge_index, W1_l, b1_l, W1_r, W2_l, b2_l, W2_r):
    x = x.astype(_f32)
    src = edge_index[0].astype(jnp.int32)
    dst = edge_index[1].astype(jnp.int32)

    n_pad = E_PAD - N_EDGES
    pad_ar = jnp.arange(n_pad, dtype=jnp.int32)
    src_pad = (pad_ar * 131) % N_NODES
    dst_pad = N_NODES + pad_ar % (NPAD - N_NODES)
    src2d = jnp.concatenate([src, src_pad]).reshape(NCH_TOT, CH)
    dst2d = jnp.concatenate([dst, dst_pad]).reshape(NCH_TOT, CH)

    z2d = jnp.zeros((NPAD, D), _f32)
    z1d = jnp.zeros((DEG_E,), _f32)

    aggp1, deg_flat = _sc_agg_deg(x, src2d, dst2d, z2d, z1d)
    degp_t = deg_flat.reshape(NC, NPAD)[:, :N_NODES].T
    h, inv = _tc1(aggp1, degp_t, x, W1_l, W1_r, b1_l.reshape(1, D))

    (aggp2,) = _sc_agg(h, src2d, dst2d, z2d, z1d)
    out = _tc2(aggp2, inv, h, W2_l, W2_r, b2_l.reshape(1, D))
    return out
